# Initial kernel scaffold; baseline (speedup 1.0000x reference)
import jax
import jax.numpy as jnp
from jax.experimental import pallas as pl


def kernel(evolved_tokens, token_batch, attn_weights, var_types, z_var_0, var_batch, variable_features, params):
    raise NotImplementedError("write your pallas kernel here")


# speedup vs baseline: 28.3726x; 28.3726x over previous
import functools

import jax
import jax.numpy as jnp
from jax.experimental import pallas as pl

B = 8
K = 64
EMB = 256
F = 23
LB_COL = 21
UB_COL = 22
INF_THRESHOLD = 1e18
THRESH = 10
NCLS = THRESH + 1

BLOCK_N = 1000


def _ln(x, g, b):
    m = x.mean(-1, keepdims=True)
    v = ((x - m) ** 2).mean(-1, keepdims=True)
    return (x - m) * jax.lax.rsqrt(v + 1e-5) * g + b


def _head_block(x, ng, nb, w1, b1, w2, b2, wh, bh):
    h = _ln(x, ng, nb)
    h = jax.nn.gelu(h @ w1 + b1)
    h = jax.nn.gelu(h @ w2 + b2)
    h = x + h
    return h @ wh + bh


def _fused_kernel(
    tokens_ref, attn_ref, vb_ref, vt_ref, z0_ref, vf_ref,
    dw_ref, db_ref, fg_ref, fb_ref,
    bin_ng, bin_nb, bin_w1, bin_b1, bin_w2, bin_b2, bin_wh, bin_bh,
    int_ng, int_nb, int_w1, int_b1, int_w2, int_b2, int_wh, int_bh,
    lrg_ng, lrg_nb, lrg_w1, lrg_b1, lrg_w2, lrg_b2, lrg_wh, lrg_bh,
    zout_ref, pbin_ref, lsmall_ref, plarge_ref,
):
    attn = attn_ref[...]
    vb = vb_ref[...]
    col_batch = jax.lax.broadcasted_iota(jnp.int32, (BLOCK_N, B * K), 1) // K
    attn_tiled = jnp.concatenate([attn] * B, axis=1)
    P = jnp.where(col_batch == vb, attn_tiled, 0.0)
    z = jnp.dot(P, tokens_ref[...], preferred_element_type=jnp.float32)
    z = jnp.dot(z, dw_ref[...], preferred_element_type=jnp.float32) + db_ref[...]
    z_out = _ln(z + z0_ref[...], fg_ref[...], fb_ref[...])
    zout_ref[...] = z_out

    vt = vt_ref[...]
    lb = vf_ref[:, LB_COL][:, None]
    ub = vf_ref[:, UB_COL][:, None]
    is_int = vt == 2
    finite = (jnp.abs(lb) < INF_THRESHOLD) & (jnp.abs(ub) < INF_THRESHOLD)
    mask_small = is_int & finite & ((ub - lb) <= THRESH)
    mask_large = is_int & (~mask_small)
    mask_bin = vt == 1
    ranges = jnp.clip((jnp.ceil(ub) - jnp.floor(lb) + 1).astype(jnp.int32), 1, NCLS)

    out_bin = _head_block(z_out, bin_ng[...], bin_nb[...], bin_w1[...], bin_b1[...],
                          bin_w2[...], bin_b2[...], bin_wh[...], bin_bh[...])
    pbin_ref[...] = jax.nn.sigmoid(out_bin) * mask_bin.astype(jnp.float32)

    logits = _head_block(z_out, int_ng[...], int_nb[...], int_w1[...], int_b1[...],
                         int_w2[...], int_b2[...], int_wh[...], int_bh[...])
    valid = jax.lax.broadcasted_iota(jnp.int32, (BLOCK_N, NCLS), 1) < ranges
    logits = jnp.where(valid, logits, -1e9)
    lsmall_ref[...] = jnp.where(mask_small, logits, 0.0)

    out_lrg = _head_block(z_out, lrg_ng[...], lrg_nb[...], lrg_w1[...], lrg_b1[...],
                          lrg_w2[...], lrg_b2[...], lrg_wh[...], lrg_bh[...])
    plarge_ref[...] = out_lrg * mask_large.astype(jnp.float32)


def _row(i):
    return (i, 0)


def _full(i):
    return (0, 0)


def _full1(i):
    return (0,)


@jax.jit
def kernel(evolved_tokens, token_batch, attn_weights, var_types, z_var_0,
           var_batch, variable_features, params):
    n = attn_weights.shape[0]
    grid = (n // BLOCK_N,)
    vb2 = var_batch.astype(jnp.int32)[:, None]
    vt2 = var_types.astype(jnp.int32)[:, None]

    def head_specs():
        return [
            pl.BlockSpec((EMB,), _full1),
            pl.BlockSpec((EMB,), _full1),
            pl.BlockSpec((EMB, EMB), _full),
            pl.BlockSpec((EMB,), _full1),
            pl.BlockSpec((EMB, EMB), _full),
            pl.BlockSpec((EMB,), _full1),
            None,
            None,
        ]

    def head_args(p):
        return [p['ng'], p['nb'], p['w1'], p['b1'], p['w2'], p['b2'], p['wh'], p['bh']]

    bin_specs = head_specs()
    bin_specs[6] = pl.BlockSpec((EMB, 1), _full)
    bin_specs[7] = pl.BlockSpec((1,), _full1)
    int_specs = head_specs()
    int_specs[6] = pl.BlockSpec((EMB, NCLS), _full)
    int_specs[7] = pl.BlockSpec((NCLS,), _full1)
    lrg_specs = head_specs()
    lrg_specs[6] = pl.BlockSpec((EMB, 1), _full)
    lrg_specs[7] = pl.BlockSpec((1,), _full1)

    in_specs = [
        pl.BlockSpec((B * K, EMB), _full),
        pl.BlockSpec((BLOCK_N, K), _row),
        pl.BlockSpec((BLOCK_N, 1), _row),
        pl.BlockSpec((BLOCK_N, 1), _row),
        pl.BlockSpec((BLOCK_N, EMB), _row),
        pl.BlockSpec((BLOCK_N, F), _row),
        pl.BlockSpec((EMB, EMB), _full),
        pl.BlockSpec((EMB,), _full1),
        pl.BlockSpec((EMB,), _full1),
        pl.BlockSpec((EMB,), _full1),
    ] + bin_specs + int_specs + lrg_specs

    out_specs = [
        pl.BlockSpec((BLOCK_N, EMB), _row),
        pl.BlockSpec((BLOCK_N, 1), _row),
        pl.BlockSpec((BLOCK_N, NCLS), _row),
        pl.BlockSpec((BLOCK_N, 1), _row),
    ]
    out_shape = [
        jax.ShapeDtypeStruct((n, EMB), jnp.float32),
        jax.ShapeDtypeStruct((n, 1), jnp.float32),
        jax.ShapeDtypeStruct((n, NCLS), jnp.float32),
        jax.ShapeDtypeStruct((n, 1), jnp.float32),
    ]

    args = [evolved_tokens, attn_weights, vb2, vt2, z_var_0, variable_features,
            params['deslice_w'], params['deslice_b'], params['fus_g'], params['fus_b']]
    args += head_args(params['bin']) + head_args(params['ints']) + head_args(params['intl'])

    z_out, prob_bin, logits_int_small, pred_int_large = pl.pallas_call(
        _fused_kernel,
        grid=grid,
        in_specs=in_specs,
        out_specs=out_specs,
        out_shape=out_shape,
    )(*args)
    return (z_out, prob_bin, logits_int_small, pred_int_large)
